# baseline (speedup 1.0000x reference)
import jax
import jax.numpy as jnp
from jax.experimental import pallas as pl
from jax.experimental.pallas import tpu as pltpu

_SB = 64


def kernel(x, pe_weight):
    batch = x.shape[0]
    max_len, d_model = pe_weight.shape
    sb = _SB if batch % _SB == 0 else 1
    n_copies = batch // sb

    def body(pe_ref, out_ref, scratch_ref, sem):
        scratch_ref[...] = jnp.broadcast_to(
            pe_ref[...][None, :, :], scratch_ref.shape
        )
        copies = [
            pltpu.make_async_copy(
                scratch_ref, out_ref.at[pl.ds(i * sb, sb)], sem
            )
            for i in range(n_copies)
        ]
        for c in copies:
            c.start()
        for c in copies:
            c.wait()

    return pl.pallas_call(
        body,
        in_specs=[pl.BlockSpec(memory_space=pltpu.MemorySpace.VMEM)],
        out_specs=pl.BlockSpec(memory_space=pl.ANY),
        out_shape=jax.ShapeDtypeStruct((batch, max_len, d_model), pe_weight.dtype),
        scratch_shapes=[
            pltpu.VMEM((sb, max_len, d_model), pe_weight.dtype),
            pltpu.SemaphoreType.DMA,
        ],
    )(pe_weight)
